# Initial kernel scaffold; baseline (speedup 1.0000x reference)
import jax
import jax.numpy as jnp
from jax.experimental import pallas as pl


def kernel(x, edge_index, W1, b1, W2, b2):
    raise NotImplementedError("write your pallas kernel here")


# speedup vs baseline: 39.6288x; 39.6288x over previous
import functools

import jax
import jax.numpy as jnp
from jax import lax
from jax.experimental import pallas as pl
from jax.experimental.pallas import tpu as pltpu
from jax.experimental.pallas import tpu_sc as plsc

_N = 10000
_NPAD = 10240
_E = 320000
_NC, _NS = 2, 16
_NW = _NC * _NS
_EPW = _E // _NW
_K = 80
_CH = _EPW // _K
_D = 16
_RS = _NPAD // _NS

_mesh = plsc.VectorSubcoreMesh(
    core_axis_name="c", subcore_axis_name="s",
    num_cores=_NC, num_subcores=_NS)
_SC_PARAMS = pltpu.CompilerParams(use_tc_tiling_on_sc=False)


@functools.partial(
    pl.kernel,
    out_type=jax.ShapeDtypeStruct((_NC, _NPAD), jnp.float32),
    mesh=_mesh, compiler_params=_SC_PARAMS,
    scratch_types=[
        pltpu.VMEM((_CH, _K), jnp.int32),
        pltpu.VMEM((_K,), jnp.float32),
        pltpu.VMEM_SHARED((_NPAD,), jnp.float32),
    ],
)
def _sc_degree(dst_hbm, zero_hbm, one_hbm, out_hbm, idx_v, ones_v, deg_sp):
    c = lax.axis_index("c")
    s = lax.axis_index("s")
    wid = s * _NC + c
    pltpu.sync_copy(zero_hbm.at[pl.ds(s * _RS, _RS)],
                    deg_sp.at[pl.ds(s * _RS, _RS)])
    pltpu.sync_copy(one_hbm, ones_v)
    pltpu.sync_copy(dst_hbm.at[wid], idx_v)
    plsc.subcore_barrier()

    def body(j, carry):
        pltpu.sync_copy(ones_v, deg_sp.at[idx_v.at[j]], add=True)
        return carry

    lax.fori_loop(0, _CH, body, 0)
    plsc.subcore_barrier()
    pltpu.sync_copy(deg_sp.at[pl.ds(s * _RS, _RS)],
                    out_hbm.at[c, pl.ds(s * _RS, _RS)])


@functools.partial(
    pl.kernel,
    out_type=jax.ShapeDtypeStruct((_NC, _NPAD, _D), jnp.float32),
    mesh=_mesh, compiler_params=_SC_PARAMS,
    scratch_types=[
        pltpu.VMEM((_CH, _K), jnp.int32),
        pltpu.VMEM((_CH, _K), jnp.int32),
        pltpu.VMEM((_K, _D), jnp.float32),
        pltpu.VMEM_SHARED((_NPAD, _D), jnp.float32),
        pltpu.VMEM_SHARED((_NPAD, _D), jnp.float32),
        pltpu.SemaphoreType.DMA,
    ],
)
def _sc_segsum(g_hbm, src_hbm, dst_hbm, zero_hbm, out_hbm,
               idx_s_v, idx_d_v, rows_v, g_sp, acc_sp, sem):
    c = lax.axis_index("c")
    s = lax.axis_index("s")
    wid = s * _NC + c
    pltpu.sync_copy(zero_hbm.at[pl.ds(s * _RS, _RS)],
                    acc_sp.at[pl.ds(s * _RS, _RS)])
    pltpu.sync_copy(g_hbm.at[pl.ds(s * _RS, _RS)],
                    g_sp.at[pl.ds(s * _RS, _RS)])
    pltpu.sync_copy(src_hbm.at[wid], idx_s_v)
    pltpu.sync_copy(dst_hbm.at[wid], idx_d_v)
    plsc.subcore_barrier()

    def body(j, carry):
        pltpu.async_copy(g_sp.at[idx_s_v.at[j]], rows_v, sem).wait()
        pltpu.sync_copy(rows_v, acc_sp.at[idx_d_v.at[j]], add=True)
        return carry

    lax.fori_loop(0, _CH, body, 0)
    plsc.subcore_barrier()
    pltpu.sync_copy(acc_sp.at[pl.ds(s * _RS, _RS)],
                    out_hbm.at[c, pl.ds(s * _RS, _RS)])


def _tc1_body(x_ref, w_ref, deg_ref, g_ref, dinv_ref):
    deg = deg_ref[0] + deg_ref[1]
    dinv = jnp.where(deg > 0, lax.rsqrt(jnp.maximum(deg, 1e-12)), 0.0)
    h = jnp.dot(x_ref[...], w_ref[...], preferred_element_type=jnp.float32)
    g_ref[...] = h * dinv
    dinv_ref[...] = dinv


def _tc2_body(acc_ref, dinv_ref, b_ref, g_ref):
    dinv = dinv_ref[...]
    u = jnp.maximum(dinv * (acc_ref[0] + acc_ref[1]) + b_ref[...], 0.0)
    g_ref[...] = dinv * u


def _tc3_body(acc_ref, dinv_ref, w_ref, b_ref, o_ref):
    t = dinv_ref[...] * (acc_ref[0] + acc_ref[1])
    o = jnp.dot(t, w_ref[...], preferred_element_type=jnp.float32) + b_ref[...]
    o = jnp.maximum(o, 0.0)
    m = jnp.max(o, axis=1, keepdims=True)
    sh = o - m
    lse = jnp.log(jnp.sum(jnp.exp(sh), axis=1, keepdims=True))
    o_ref[...] = sh - lse


def kernel(x, edge_index, W1, b1, W2, b2):
    f32 = jnp.float32
    ei = edge_index.astype(jnp.int32).reshape(2, _NW, _CH, _K)
    src3, dst3 = ei[0], ei[1]
    x_pad = jnp.pad(x.astype(f32), ((0, _NPAD - _N), (0, 0)))
    zeros1 = jnp.zeros((_NPAD,), f32)
    zeros2 = jnp.zeros((_NPAD, _D), f32)
    ones = jnp.ones((_K,), f32)

    deg2 = _sc_degree(dst3, zeros1, ones)

    g1, dinv = pl.pallas_call(
        _tc1_body,
        out_shape=[jax.ShapeDtypeStruct((_NPAD, _D), f32),
                   jax.ShapeDtypeStruct((_NPAD, 1), f32)],
    )(x_pad, W1, deg2.reshape(_NC, _NPAD, 1))

    acc1 = _sc_segsum(g1, src3, dst3, zeros2)

    g2 = pl.pallas_call(
        _tc2_body,
        out_shape=jax.ShapeDtypeStruct((_NPAD, _D), f32),
    )(acc1, dinv, b1.reshape(1, _D))

    acc2 = _sc_segsum(g2, src3, dst3, zeros2)

    out = pl.pallas_call(
        _tc3_body,
        out_shape=jax.ShapeDtypeStruct((_NPAD, W2.shape[1]), f32),
    )(acc2, dinv, W2, b2.reshape(1, W2.shape[1]))

    return out[:_N]


# speedup vs baseline: 50.6755x; 1.2788x over previous
import functools

import jax
import jax.numpy as jnp
from jax import lax
from jax.experimental import pallas as pl
from jax.experimental.pallas import tpu as pltpu
from jax.experimental.pallas import tpu_sc as plsc

_N = 10000
_NPAD = 10240
_E = 320000
_NC, _NS = 2, 16
_NW = _NC * _NS
_EPW = _E // _NW
_K = 80
_CH = _EPW // _K
_D = 16
_RS = _NPAD // _NS
_G = 5

_mesh = plsc.VectorSubcoreMesh(
    core_axis_name="c", subcore_axis_name="s",
    num_cores=_NC, num_subcores=_NS)
_SC_PARAMS = pltpu.CompilerParams(use_tc_tiling_on_sc=False)


@functools.partial(
    pl.kernel,
    out_type=jax.ShapeDtypeStruct((_NC, _NPAD), jnp.float32),
    mesh=_mesh, compiler_params=_SC_PARAMS,
    scratch_types=[
        pltpu.VMEM((_CH, _K), jnp.int32),
        pltpu.VMEM((_K,), jnp.float32),
        pltpu.VMEM_SHARED((_NPAD,), jnp.float32),
        pltpu.SemaphoreType.DMA,
    ],
)
def _sc_degree(ei_hbm, zero_hbm, one_hbm, out_hbm, idx_v, ones_v, deg_sp, sem):
    c = lax.axis_index("c")
    s = lax.axis_index("s")
    wid = s * _NC + c
    pltpu.sync_copy(zero_hbm.at[pl.ds(s * _RS, _RS)],
                    deg_sp.at[pl.ds(s * _RS, _RS)])
    pltpu.sync_copy(one_hbm, ones_v)
    pltpu.sync_copy(ei_hbm.at[1, wid], idx_v)
    plsc.subcore_barrier()

    def body(g, carry):
        for b in range(_G):
            pltpu.async_copy(ones_v, deg_sp.at[idx_v.at[g * _G + b]], sem,
                             add=True)
        for b in range(_G):
            pltpu.make_async_copy(ones_v, deg_sp.at[idx_v.at[g * _G + b]],
                                  sem).wait()
        return carry

    lax.fori_loop(0, _CH // _G, body, 0)
    plsc.subcore_barrier()
    pltpu.sync_copy(deg_sp.at[pl.ds(s * _RS, _RS)],
                    out_hbm.at[c, pl.ds(s * _RS, _RS)])


@functools.partial(
    pl.kernel,
    out_type=jax.ShapeDtypeStruct((_NC, _NPAD, _D), jnp.float32),
    mesh=_mesh, compiler_params=_SC_PARAMS,
    scratch_types=[
        pltpu.VMEM((_CH, _K), jnp.int32),
        pltpu.VMEM((_CH, _K), jnp.int32),
        pltpu.VMEM((_K, _D), jnp.float32),
        pltpu.VMEM((_K, _D), jnp.float32),
        pltpu.VMEM_SHARED((_NPAD, _D), jnp.float32),
        pltpu.VMEM_SHARED((_NPAD, _D), jnp.float32),
        pltpu.SemaphoreType.DMA,
        pltpu.SemaphoreType.DMA,
    ],
)
def _sc_segsum(g_hbm, ei_hbm, zero_hbm, out_hbm,
               idx_s_v, idx_d_v, rows_a, rows_b, g_sp, acc_sp, sem_a, sem_b):
    c = lax.axis_index("c")
    s = lax.axis_index("s")
    wid = s * _NC + c
    pltpu.sync_copy(zero_hbm.at[pl.ds(s * _RS, _RS)],
                    acc_sp.at[pl.ds(s * _RS, _RS)])
    pltpu.sync_copy(g_hbm.at[pl.ds(s * _RS, _RS)],
                    g_sp.at[pl.ds(s * _RS, _RS)])
    pltpu.sync_copy(ei_hbm.at[0, wid], idx_s_v)
    pltpu.sync_copy(ei_hbm.at[1, wid], idx_d_v)
    plsc.subcore_barrier()

    pltpu.async_copy(g_sp.at[idx_s_v.at[0]], rows_a, sem_a)

    def body(p, carry):
        c0 = 2 * p
        pltpu.async_copy(g_sp.at[idx_s_v.at[c0 + 1]], rows_b, sem_b)
        pltpu.make_async_copy(g_sp.at[idx_s_v.at[c0]], rows_a, sem_a).wait()
        pltpu.sync_copy(rows_a, acc_sp.at[idx_d_v.at[c0]], add=True)
        pltpu.async_copy(g_sp.at[idx_s_v.at[c0 + 2]], rows_a, sem_a)
        pltpu.make_async_copy(g_sp.at[idx_s_v.at[c0 + 1]], rows_b, sem_b).wait()
        pltpu.sync_copy(rows_b, acc_sp.at[idx_d_v.at[c0 + 1]], add=True)
        return carry

    lax.fori_loop(0, _CH // 2, body, 0)
    pltpu.make_async_copy(g_sp.at[idx_s_v.at[_CH - 1]], rows_a, sem_a).wait()
    pltpu.sync_copy(rows_a, acc_sp.at[idx_d_v.at[_CH - 1]], add=True)
    plsc.subcore_barrier()
    pltpu.sync_copy(acc_sp.at[pl.ds(s * _RS, _RS)],
                    out_hbm.at[c, pl.ds(s * _RS, _RS)])


def _tc1a_body(x_ref, w_ref, h_ref):
    h_ref[_N:, :] = jnp.zeros((_NPAD - _N, _D), jnp.float32)
    h_ref[:_N, :] = jnp.dot(x_ref[...], w_ref[...],
                            preferred_element_type=jnp.float32)


def _tc1b_body(h_ref, deg_ref, g_ref, dinv_ref):
    deg = deg_ref[0] + deg_ref[1]
    dinv = jnp.where(deg > 0, lax.rsqrt(jnp.maximum(deg, 1e-12)), 0.0)
    g_ref[...] = h_ref[...] * dinv
    dinv_ref[...] = dinv


def _tc2_body(acc_ref, dinv_ref, b_ref, g_ref):
    dinv = dinv_ref[...]
    u = jnp.maximum(dinv * (acc_ref[0] + acc_ref[1]) + b_ref[...], 0.0)
    g_ref[...] = dinv * u


def _tc3_body(acc_ref, dinv_ref, w_ref, b_ref, o_ref):
    t = dinv_ref[:_N] * (acc_ref[0, :_N] + acc_ref[1, :_N])
    o = jnp.dot(t, w_ref[...], preferred_element_type=jnp.float32) + b_ref[...]
    o = jnp.maximum(o, 0.0)
    m = jnp.max(o, axis=1, keepdims=True)
    sh = o - m
    lse = jnp.log(jnp.sum(jnp.exp(sh), axis=1, keepdims=True))
    o_ref[...] = sh - lse


def kernel(x, edge_index, W1, b1, W2, b2):
    f32 = jnp.float32
    ei3 = edge_index.astype(jnp.int32).reshape(2, _NW, _CH, _K)
    zeros1 = jnp.zeros((_NPAD,), f32)
    zeros2 = jnp.zeros((_NPAD, _D), f32)
    ones = jnp.ones((_K,), f32)

    deg2 = _sc_degree(ei3, zeros1, ones)

    h1 = pl.pallas_call(
        _tc1a_body,
        out_shape=jax.ShapeDtypeStruct((_NPAD, _D), f32),
    )(x, W1)

    g1, dinv = pl.pallas_call(
        _tc1b_body,
        out_shape=[jax.ShapeDtypeStruct((_NPAD, _D), f32),
                   jax.ShapeDtypeStruct((_NPAD, 1), f32)],
    )(h1, deg2.reshape(_NC, _NPAD, 1))

    acc1 = _sc_segsum(g1, ei3, zeros2)

    g2 = pl.pallas_call(
        _tc2_body,
        out_shape=jax.ShapeDtypeStruct((_NPAD, _D), f32),
    )(acc1, dinv, b1.reshape(1, _D))

    acc2 = _sc_segsum(g2, ei3, zeros2)

    out = pl.pallas_call(
        _tc3_body,
        out_shape=jax.ShapeDtypeStruct((_N, W2.shape[1]), f32),
    )(acc2, dinv, W2, b2.reshape(1, W2.shape[1]))

    return out
